# baseline (speedup 1.0000x reference)
import functools

import jax
import jax.numpy as jnp
from jax import lax
from jax.experimental import pallas as pl
from jax.experimental.pallas import tpu as pltpu
from jax.experimental.pallas import tpu_sc as plsc

B = 16
L = 4096
TOKEN_DIM = 64
STRIDE = 56
NT = 73
W0 = 40
W1 = NT - W0
IN0 = (W0 - 1) * STRIDE + TOKEN_DIM
IN1 = (W1 - 1) * STRIDE + TOKEN_DIM
LANES = 16

_mesh = plsc.VectorSubcoreMesh(core_axis_name="c", subcore_axis_name="s")


@functools.partial(
    pl.kernel,
    mesh=_mesh,
    out_type=jax.ShapeDtypeStruct((B, NT, TOKEN_DIM), jnp.float32),
    scratch_types=[
        pltpu.VMEM((IN0,), jnp.float32),
        pltpu.VMEM((W0, TOKEN_DIM), jnp.float32),
    ],
)
def _tokenize_sc(in_hbm, out_hbm, in_v, out_v):
    row = lax.axis_index("s")
    half = lax.axis_index("c")

    @pl.when(half == 0)
    def _():
        in_off = pl.multiple_of(row * L, 8)
        pltpu.sync_copy(in_hbm.at[pl.ds(in_off, IN0)], in_v)
        for t in range(W0):
            for j in range(TOKEN_DIM // LANES):
                out_v[t, pl.ds(j * LANES, LANES)] = in_v[
                    pl.ds(t * STRIDE + j * LANES, LANES)
                ]
        pltpu.sync_copy(out_v, out_hbm.at[row, pl.ds(0, W0)])

    @pl.when(half == 1)
    def _():
        in_off = pl.multiple_of(row * L + W0 * STRIDE, 8)
        pltpu.sync_copy(in_hbm.at[pl.ds(in_off, IN1)], in_v.at[pl.ds(0, IN1)])
        for t in range(W1):
            for j in range(TOKEN_DIM // LANES):
                out_v[t, pl.ds(j * LANES, LANES)] = in_v[
                    pl.ds(t * STRIDE + j * LANES, LANES)
                ]
        pltpu.sync_copy(out_v.at[pl.ds(0, W1)], out_hbm.at[row, pl.ds(W0, W1)])


def kernel(inputs):
    return _tokenize_sc(inputs.reshape(B * L))


# speedup vs baseline: 5.3951x; 5.3951x over previous
import jax
import jax.numpy as jnp
from jax import lax
from jax.experimental import pallas as pl

B = 16
L = 4096
TOKEN_DIM = 64
STRIDE = 56
NT = 73
OUT_W = NT * TOKEN_DIM
FULL_TILES = OUT_W // 128


def _tokenize_tc_body(in_ref, out_ref):
    lane = lax.broadcasted_iota(jnp.int32, (B, 128), 1)
    first_half = lane < TOKEN_DIM
    for k in range(FULL_TILES):
        a = in_ref[:, 112 * k:112 * k + 128]
        if k == 0:
            b = jnp.roll(a, 8, axis=1)
        else:
            b = in_ref[:, 112 * k - 8:112 * k + 120]
        out_ref[:, 128 * k:128 * k + 128] = jnp.where(first_half, a, b)
    out_ref[:, FULL_TILES * 128:] = in_ref[:, STRIDE * (NT - 1):]


def kernel(inputs):
    flat = pl.pallas_call(
        _tokenize_tc_body,
        out_shape=jax.ShapeDtypeStruct((B, OUT_W), jnp.float32),
    )(inputs)
    return flat.reshape(B, NT, TOKEN_DIM)
